# Initial kernel scaffold; baseline (speedup 1.0000x reference)
import jax
import jax.numpy as jnp
from jax.experimental import pallas as pl


def kernel(x, weight):
    raise NotImplementedError("write your pallas kernel here")


# speedup vs baseline: 1.5725x; 1.5725x over previous
import functools

import jax
import jax.numpy as jnp
from jax import lax
from jax.experimental import pallas as pl
from jax.experimental.pallas import tpu as pltpu
from jax.experimental.pallas import tpu_sc as plsc

_CH = 128


@functools.lru_cache(maxsize=None)
def _make_gather(n_chunks: int, d: int, n_cores: int, n_subcores: int):
    n_workers = n_cores * n_subcores
    chunks_per_w = n_chunks // n_workers
    mesh = plsc.VectorSubcoreMesh(core_axis_name="c", subcore_axis_name="s")

    @functools.partial(
        pl.kernel,
        mesh=mesh,
        out_type=jax.ShapeDtypeStruct((n_chunks * _CH, d), jnp.float32),
        scratch_types=[
            pltpu.VMEM((_CH,), jnp.int32),
            pltpu.VMEM((_CH, d), jnp.float32),
            pltpu.SemaphoreType.DMA,
        ],
        compiler_params=pltpu.CompilerParams(use_tc_tiling_on_sc=False),
    )
    def k(idx_hbm, table_hbm, out_hbm, idx_v, rows_v, sem):
        wid = lax.axis_index("s") * n_cores + lax.axis_index("c")

        def step(i, carry):
            chunk = wid * chunks_per_w + i
            pltpu.sync_copy(idx_hbm.at[chunk], idx_v)
            pltpu.async_copy(table_hbm.at[idx_v], rows_v, sem).wait()
            pltpu.sync_copy(rows_v, out_hbm.at[pl.ds(chunk * _CH, _CH)])
            return carry

        lax.fori_loop(0, chunks_per_w, step, 0)

    return k


def kernel(x, weight):
    b, s = x.shape
    _, d = weight.shape
    n = b * s
    info = plsc.get_sparse_core_info()
    nc, ns = info.num_cores, info.num_subcores
    assert n % (_CH * nc * ns) == 0
    n_chunks = n // _CH
    out = _make_gather(n_chunks, d, nc, ns)(x.reshape(n_chunks, _CH), weight)
    return out.reshape(b, s, d)


# speedup vs baseline: 1.8807x; 1.1960x over previous
import functools

import jax
import jax.numpy as jnp
from jax import lax
from jax.experimental import pallas as pl
from jax.experimental.pallas import tpu as pltpu
from jax.experimental.pallas import tpu_sc as plsc

_CH = 128
_NBUF = 4


@functools.lru_cache(maxsize=None)
def _make_gather(n_chunks: int, d: int, n_cores: int, n_subcores: int):
    n_workers = n_cores * n_subcores
    chunks_per_w = n_chunks // n_workers
    n_groups = chunks_per_w // _NBUF
    assert chunks_per_w % _NBUF == 0
    mesh = plsc.VectorSubcoreMesh(core_axis_name="c", subcore_axis_name="s")

    @functools.partial(
        pl.kernel,
        mesh=mesh,
        out_type=jax.ShapeDtypeStruct((n_chunks * _CH, d), jnp.float32),
        scratch_types=[
            pltpu.VMEM((chunks_per_w, _CH), jnp.int32),
            *[pltpu.VMEM((_CH, d), jnp.float32) for _ in range(_NBUF)],
            *[pltpu.SemaphoreType.DMA for _ in range(2 * _NBUF)],
        ],
        compiler_params=pltpu.CompilerParams(use_tc_tiling_on_sc=False),
    )
    def k(idx_hbm, table_hbm, out_hbm, idx_v, *bufs_and_sems):
        rows = bufs_and_sems[:_NBUF]
        gsem = bufs_and_sems[_NBUF:2 * _NBUF]
        wsem = bufs_and_sems[2 * _NBUF:]
        wid = lax.axis_index("s") * n_cores + lax.axis_index("c")
        c0 = wid * chunks_per_w
        pltpu.sync_copy(idx_hbm.at[pl.ds(c0, chunks_per_w)], idx_v)

        def gather_start(i, b):
            pltpu.async_copy(table_hbm.at[idx_v.at[i]], rows[b], gsem[b])

        def gather_wait(i, b):
            pltpu.make_async_copy(table_hbm.at[idx_v.at[i]], rows[b],
                                  gsem[b]).wait()

        def wb_start(i, b):
            pltpu.async_copy(rows[b], out_hbm.at[pl.ds((c0 + i) * _CH, _CH)],
                             wsem[b])

        def wb_wait(i, b):
            pltpu.make_async_copy(rows[b],
                                  out_hbm.at[pl.ds((c0 + i) * _CH, _CH)],
                                  wsem[b]).wait()

        for b in range(_NBUF):
            gather_start(b, b)

        def group(g, carry):
            for b in range(_NBUF):
                i = g * _NBUF + b
                gather_wait(i, b)
                wb_start(i, b)

                @pl.when(g < n_groups - 1)
                def _prefetch(i=i, b=b):
                    wb_wait(i, b)
                    gather_start(i + _NBUF, b)
            return carry

        lax.fori_loop(0, n_groups, group, 0)
        for b in range(_NBUF):
            wb_wait((n_groups - 1) * _NBUF + b, b)

    return k


def kernel(x, weight):
    b, s = x.shape
    _, d = weight.shape
    n = b * s
    info = plsc.get_sparse_core_info()
    nc, ns = info.num_cores, info.num_subcores
    assert n % (_CH * nc * ns) == 0
    n_chunks = n // _CH
    out = _make_gather(n_chunks, d, nc, ns)(x.reshape(n_chunks, _CH), weight)
    return out.reshape(b, s, d)
